# Initial kernel scaffold; baseline (speedup 1.0000x reference)
import jax
import jax.numpy as jnp
from jax.experimental import pallas as pl


def kernel(x, coords, coord_weights, sub_layer_weights, W0, W1):
    raise NotImplementedError("write your pallas kernel here")


# speedup vs baseline: 1.6556x; 1.6556x over previous
import functools

import jax
import jax.numpy as jnp
from jax import lax
from jax.experimental import pallas as pl
from jax.experimental.pallas import tpu as pltpu
from jax.experimental.pallas import tpu_sc as plsc

_T = 100000
_B = 16384
_D = 64
_I = 128
_X = 2
_NSUB = 2

_NC = 2
_NS = 16
_NW = _NC * _NS
_SUB = 128
_NE = _B * _X * _NSUB
_K = _NE // (_NW * _SUB)


def _sc_gather_body(idx_hbm, table_hbm, out_hbm, tab_sp, idx_v, rows_v, sem):
    sid = lax.axis_index("s")
    wid = sid * _NC + lax.axis_index("c")
    @pl.when(sid == 0)
    def _():
        pltpu.sync_copy(table_hbm, tab_sp)

    pltpu.sync_copy(idx_hbm.at[wid], idx_v)
    plsc.subcore_barrier()
    copies = []
    for j in range(_K):
        copies.append(
            pltpu.make_async_copy(tab_sp.at[idx_v.at[j]], rows_v.at[j], sem)
        )
    for c in copies:
        c.start()
    for c in copies:
        c.wait()
    pltpu.sync_copy(rows_v, out_hbm.at[wid])


@jax.jit
def _sc_gather(idx, table_flat):
    mesh = plsc.VectorSubcoreMesh(core_axis_name="c", subcore_axis_name="s")
    fn = functools.partial(
        pl.kernel,
        mesh=mesh,
        out_type=jax.ShapeDtypeStruct((_NW, _K, _SUB), jnp.float32),
        scratch_types=[
            pltpu.VMEM_SHARED((_T * _NSUB,), jnp.float32),
            pltpu.VMEM((_K, _SUB), jnp.int32),
            pltpu.VMEM((_K, _SUB), jnp.float32),
            pltpu.SemaphoreType.DMA,
        ],
    )(_sc_gather_body)
    return fn(idx, table_flat)


def _combine_body(vals_ref, cw_ref, x_ref, w_ref, o_ref):
    v = vals_ref[...]
    sp = jnp.maximum(v, 0.0) + jnp.log1p(jnp.exp(-jnp.abs(v)))
    cw = cw_ref[...]
    w0 = sp[:, 0:1] * cw[:, 0:1] + sp[:, 2:3] * cw[:, 1:2]
    w1 = sp[:, 1:2] * cw[:, 0:1] + sp[:, 3:4] * cw[:, 1:2]
    xb = x_ref[...]
    xs = jnp.concatenate([xb * w0, xb * w1], axis=1)
    o_ref[...] = jnp.dot(xs, w_ref[...], preferred_element_type=jnp.float32,
                         precision=lax.Precision.HIGHEST)


def _tc_combine(vals, cw, x, wcat, blk):
    nblk = _B // blk
    return pl.pallas_call(
        _combine_body,
        grid=(nblk,),
        in_specs=[
            pl.BlockSpec((blk, _X * _NSUB), lambda i: (i, 0)),
            pl.BlockSpec((blk, _X), lambda i: (i, 0)),
            pl.BlockSpec((blk, _D), lambda i: (i, 0)),
            pl.BlockSpec((2 * _D, _I), lambda i: (0, 0)),
        ],
        out_specs=pl.BlockSpec((blk, _I), lambda i: (i, 0)),
        out_shape=jax.ShapeDtypeStruct((_B, _I), jnp.float32),
    )(vals, cw, x, wcat)


def kernel(x, coords, coord_weights, sub_layer_weights, W0, W1):
    c = coords.reshape(_B * _X).astype(jnp.int32)
    idx = (c[:, None] * _NSUB + jnp.arange(_NSUB, dtype=jnp.int32)[None, :])
    idx = idx.reshape(_NW, _K, _SUB)
    vals = _sc_gather(idx, sub_layer_weights.reshape(_T * _NSUB))
    vals = vals.reshape(_B, _X * _NSUB)
    wcat = jnp.concatenate([W0, W1], axis=0)
    return _tc_combine(vals, coord_weights, x, wcat, blk=1024)


# speedup vs baseline: 4.5601x; 2.7543x over previous
import functools

import jax
import jax.numpy as jnp
from jax import lax
from jax.experimental import pallas as pl
from jax.experimental.pallas import tpu as pltpu
from jax.experimental.pallas import tpu_sc as plsc

_T = 100000
_B = 16384
_D = 64
_I = 128
_X = 2
_NSUB = 2

_NC = 2
_NS = 16
_NW = _NC * _NS
_SUB = 128
_NE = _B * _X * _NSUB
_K = _NE // (_NW * _SUB)


def _sc_gather_body(idx_hbm, table_hbm, out_hbm, tab_sp, idx_v, rows_v, sem):
    sid = lax.axis_index("s")
    wid = sid * _NC + lax.axis_index("c")
    @pl.when(sid == 0)
    def _():
        pltpu.sync_copy(table_hbm, tab_sp)

    pltpu.sync_copy(idx_hbm.at[wid], idx_v)
    plsc.subcore_barrier()
    copies = []
    for j in range(_K):
        copies.append(
            pltpu.make_async_copy(tab_sp.at[idx_v.at[j]], rows_v.at[j], sem)
        )
    for c in copies:
        c.start()
    for c in copies:
        c.wait()
    pltpu.sync_copy(rows_v, out_hbm.at[wid])


@jax.jit
def _sc_gather(idx, table_flat):
    mesh = plsc.VectorSubcoreMesh(core_axis_name="c", subcore_axis_name="s")
    fn = functools.partial(
        pl.kernel,
        mesh=mesh,
        out_type=jax.ShapeDtypeStruct((_NW, _K, _SUB), jnp.float32),
        scratch_types=[
            pltpu.VMEM_SHARED((_T * _NSUB,), jnp.float32),
            pltpu.VMEM((_K, _SUB), jnp.int32),
            pltpu.VMEM((_K, _SUB), jnp.float32),
            pltpu.SemaphoreType.DMA,
        ],
    )(_sc_gather_body)
    return fn(idx, table_flat)


def _combine_body(vals_ref, cw_ref, xT_ref, w_ref, o_ref):
    v = vals_ref[...]
    sp = jnp.maximum(v, 0.0) + jnp.log1p(jnp.exp(-jnp.abs(v)))
    cw = cw_ref[...]
    w0 = sp[0:1, :] * cw[0:1, :] + sp[2:3, :] * cw[1:2, :]
    w1 = sp[1:2, :] * cw[0:1, :] + sp[3:4, :] * cw[1:2, :]
    xT = xT_ref[...]
    xs = jnp.concatenate([xT * w0, xT * w1], axis=0)
    o_ref[...] = jnp.dot(w_ref[...], xs, preferred_element_type=jnp.float32,
                         precision=lax.Precision.HIGHEST)


def _tc_combine(vals, cwT, xT, wcatT, blk):
    nblk = _B // blk
    return pl.pallas_call(
        _combine_body,
        grid=(nblk,),
        in_specs=[
            pl.BlockSpec((_X * _NSUB, blk), lambda i: (0, i)),
            pl.BlockSpec((_X, blk), lambda i: (0, i)),
            pl.BlockSpec((_D, blk), lambda i: (0, i)),
            pl.BlockSpec((_I, 2 * _D), lambda i: (0, 0)),
        ],
        out_specs=pl.BlockSpec((_I, blk), lambda i: (0, i)),
        out_shape=jax.ShapeDtypeStruct((_I, _B), jnp.float32),
    )(vals, cwT, xT, wcatT)


def kernel(x, coords, coord_weights, sub_layer_weights, W0, W1):
    xT = x.T
    cwT = coord_weights.T
    cT = coords.reshape(_B, _X).T.astype(jnp.int32)
    flat_t = sub_layer_weights.T.reshape(_T * _NSUB)
    c0, c1 = cT[0], cT[1]
    idx4 = jnp.stack([c0, _T + c0, c1, _T + c1])
    idx = idx4.reshape(_NW, _K, _SUB)
    vals = _sc_gather(idx, flat_t).reshape(_X * _NSUB, _B)
    wcatT = jnp.concatenate([W0, W1], axis=0).T
    oT = _tc_combine(vals, cwT, xT, wcatT, blk=1024)
    return oT.T


# speedup vs baseline: 6.0743x; 1.3320x over previous
import functools

import jax
import jax.numpy as jnp
from jax import lax
from jax.experimental import pallas as pl
from jax.experimental.pallas import tpu as pltpu
from jax.experimental.pallas import tpu_sc as plsc

_T = 100000
_B = 16384
_D = 64
_I = 128
_X = 2
_NSUB = 2

_NC = 2
_NS = 16
_NW = _NC * _NS
_SUB = 128
_NE = _B * _X * _NSUB
_K = _NE // (_NW * _SUB)


def _sc_gather_body(idx_hbm, table_hbm, out_hbm, tab_sp, idx_v, rows_v, sem):
    sid = lax.axis_index("s")
    wid = sid * _NC + lax.axis_index("c")
    @pl.when(sid == 0)
    def _():
        pltpu.sync_copy(table_hbm, tab_sp)

    pltpu.sync_copy(idx_hbm.at[wid], idx_v)
    plsc.subcore_barrier()
    copies = []
    for j in range(_K):
        copies.append(
            pltpu.make_async_copy(tab_sp.at[idx_v.at[j]], rows_v.at[j], sem)
        )
    for c in copies:
        c.start()
    for c in copies:
        c.wait()
    pltpu.sync_copy(rows_v, out_hbm.at[wid])


@jax.jit
def _sc_gather(idx, table_flat):
    mesh = plsc.VectorSubcoreMesh(core_axis_name="c", subcore_axis_name="s")
    fn = functools.partial(
        pl.kernel,
        mesh=mesh,
        out_type=jax.ShapeDtypeStruct((_NW, _K, _SUB), jnp.float32),
        scratch_types=[
            pltpu.VMEM_SHARED((_T * _NSUB,), jnp.float32),
            pltpu.VMEM((_K, _SUB), jnp.int32),
            pltpu.VMEM((_K, _SUB), jnp.float32),
            pltpu.SemaphoreType.DMA,
        ],
    )(_sc_gather_body)
    return fn(idx, table_flat)


def _combine_body(vals_ref, cw_ref, xT_ref, w_ref, o_ref):
    v = vals_ref[...]
    sp = jnp.maximum(v, 0.0) + jnp.log1p(jnp.exp(-jnp.abs(v)))
    cw = cw_ref[...]
    w0 = sp[0:1, :] * cw[0:1, :] + sp[2:3, :] * cw[1:2, :]
    w1 = sp[1:2, :] * cw[0:1, :] + sp[3:4, :] * cw[1:2, :]
    xT = xT_ref[...]
    xs = jnp.concatenate([xT * w0, xT * w1], axis=0)
    o_ref[...] = lax.dot_general(
        xs, w_ref[...], (((0,), (0,)), ((), ())),
        preferred_element_type=jnp.float32,
        precision=lax.Precision.HIGHEST)


def _tc_combine(vals, cwT, xT, wcat, blk):
    nblk = _B // blk
    return pl.pallas_call(
        _combine_body,
        grid=(nblk,),
        in_specs=[
            pl.BlockSpec((_X * _NSUB, blk), lambda i: (0, i)),
            pl.BlockSpec((_X, blk), lambda i: (0, i)),
            pl.BlockSpec((_D, blk), lambda i: (0, i)),
            pl.BlockSpec((2 * _D, _I), lambda i: (0, 0)),
        ],
        out_specs=pl.BlockSpec((blk, _I), lambda i: (i, 0)),
        out_shape=jax.ShapeDtypeStruct((_B, _I), jnp.float32),
    )(vals, cwT, xT, wcat)


def kernel(x, coords, coord_weights, sub_layer_weights, W0, W1):
    xT = x.T
    cwT = coord_weights.T
    cT = coords.reshape(_B, _X).T.astype(jnp.int32)
    flat_t = sub_layer_weights.T.reshape(_T * _NSUB)
    c0, c1 = cT[0], cT[1]
    idx4 = jnp.stack([c0, _T + c0, c1, _T + c1])
    idx = idx4.reshape(_NW, _K, _SUB)
    vals = _sc_gather(idx, flat_t).reshape(_X * _NSUB, _B)
    wcat = jnp.concatenate([W0, W1], axis=0)
    return _tc_combine(vals, cwT, xT, wcat, blk=2048)


# speedup vs baseline: 6.5471x; 1.0778x over previous
import functools

import jax
import jax.numpy as jnp
from jax import lax
from jax.experimental import pallas as pl
from jax.experimental.pallas import tpu as pltpu
from jax.experimental.pallas import tpu_sc as plsc

_T = 100000
_B = 16384
_D = 64
_I = 128
_X = 2
_NSUB = 2

_NC = 2
_NS = 16
_NW = _NC * _NS
_SUB = 128
_NE = _B * _X * _NSUB
_K = _NE // (_NW * _SUB)


def _sc_gather_body(idx_hbm, table_hbm, out_hbm, tab_sp, idx_v, rows_v, sem):
    sid = lax.axis_index("s")
    wid = sid * _NC + lax.axis_index("c")
    @pl.when(sid == 0)
    def _():
        pltpu.sync_copy(table_hbm, tab_sp)

    pltpu.sync_copy(idx_hbm.at[wid], idx_v)
    plsc.subcore_barrier()
    copies = []
    for j in range(_K):
        copies.append(
            pltpu.make_async_copy(tab_sp.at[idx_v.at[j]], rows_v.at[j], sem)
        )
    for c in copies:
        c.start()
    for c in copies:
        c.wait()
    pltpu.sync_copy(rows_v, out_hbm.at[wid])


@jax.jit
def _sc_gather(idx, table_flat):
    mesh = plsc.VectorSubcoreMesh(core_axis_name="c", subcore_axis_name="s")
    fn = functools.partial(
        pl.kernel,
        mesh=mesh,
        out_type=jax.ShapeDtypeStruct((_NW, _K, _SUB), jnp.float32),
        scratch_types=[
            pltpu.VMEM_SHARED((_T * _NSUB,), jnp.float32),
            pltpu.VMEM((_K, _SUB), jnp.int32),
            pltpu.VMEM((_K, _SUB), jnp.float32),
            pltpu.SemaphoreType.DMA,
        ],
    )(_sc_gather_body)
    return fn(idx, table_flat)


def _combine_body(vals_ref, cw_ref, xT_ref, w_ref, o_ref):
    v = vals_ref[...]
    sp = jnp.maximum(v, 0.0) + jnp.log1p(jnp.exp(-jnp.abs(v)))
    cw = cw_ref[...]
    w0 = sp[0:1, :] * cw[0:1, :] + sp[2:3, :] * cw[1:2, :]
    w1 = sp[1:2, :] * cw[0:1, :] + sp[3:4, :] * cw[1:2, :]
    xT = xT_ref[...]
    xs = jnp.concatenate([xT * w0, xT * w1], axis=0)
    o_ref[...] = lax.dot_general(
        xs, w_ref[...], (((0,), (0,)), ((), ())),
        preferred_element_type=jnp.float32)


def _tc_combine(vals, cwT, xT, wcat, blk):
    nblk = _B // blk
    return pl.pallas_call(
        _combine_body,
        grid=(nblk,),
        in_specs=[
            pl.BlockSpec((_X * _NSUB, blk), lambda i: (0, i)),
            pl.BlockSpec((_X, blk), lambda i: (0, i)),
            pl.BlockSpec((_D, blk), lambda i: (0, i)),
            pl.BlockSpec((2 * _D, _I), lambda i: (0, 0)),
        ],
        out_specs=pl.BlockSpec((blk, _I), lambda i: (i, 0)),
        out_shape=jax.ShapeDtypeStruct((_B, _I), jnp.float32),
    )(vals, cwT, xT, wcat)


def kernel(x, coords, coord_weights, sub_layer_weights, W0, W1):
    xT = x.T
    cwT = coord_weights.T
    cT = coords.reshape(_B, _X).T.astype(jnp.int32)
    flat_t = sub_layer_weights.T.reshape(_T * _NSUB)
    c0, c1 = cT[0], cT[1]
    idx4 = jnp.stack([c0, _T + c0, c1, _T + c1])
    idx = idx4.reshape(_NW, _K, _SUB)
    vals = _sc_gather(idx, flat_t).reshape(_X * _NSUB, _B)
    wcat = jnp.concatenate([W0, W1], axis=0)
    return _tc_combine(vals, cwT, xT, wcat, blk=2048)
